# baseline (speedup 1.0000x reference)
import functools
import jax
import jax.numpy as jnp
from jax.experimental import pallas as pl

_E = 64
_TOP_K = 8
_SCALE = 2.5


def _router_block(w_ref, xa_ref, xb_ref, idx_ref, val_ref):
    w = w_ref[...]
    dn = (((1,), (1,)), ((), ()))
    la = jax.lax.dot_general(w, xa_ref[...], dimension_numbers=dn,
                             preferred_element_type=jnp.float32)
    lb = jax.lax.dot_general(w, xb_ref[...], dimension_numbers=dn,
                             preferred_element_type=jnp.float32)
    logits = jnp.concatenate([la, lb], axis=1)
    iota = jax.lax.broadcasted_iota(jnp.int32, logits.shape, 0)
    work = logits
    idx_rows = []
    val_rows = []
    for k in range(_TOP_K):
        mk = jnp.max(work, axis=0, keepdims=True)
        if k == 0:
            m = mk
            denom = jnp.sum(jnp.exp(logits - m), axis=0, keepdims=True)
            inv = _SCALE / denom
        sel = jnp.min(jnp.where(work == mk, iota, _E), axis=0, keepdims=True)
        idx_rows.append(sel)
        val_rows.append(jnp.exp(mk - m) * inv)
        work = jnp.where(iota == sel, -jnp.inf, work)
    idx_ref[...] = jnp.concatenate(idx_rows, axis=0)
    val_ref[...] = jnp.concatenate(val_rows, axis=0)


@functools.partial(jax.jit, static_argnames=("m_blk",))
def _router(flat, weight, m_blk):
    m_total, h = flat.shape
    n_steps = m_total // (2 * m_blk)
    idx_t, val_t = pl.pallas_call(
        _router_block,
        grid=(n_steps,),
        in_specs=[
            pl.BlockSpec((_E, h), lambda i: (0, 0)),
            pl.BlockSpec((m_blk, h), lambda i: (2 * i, 0)),
            pl.BlockSpec((m_blk, h), lambda i: (2 * i + 1, 0)),
        ],
        out_specs=[
            pl.BlockSpec((_TOP_K, 2 * m_blk), lambda i: (0, i)),
            pl.BlockSpec((_TOP_K, 2 * m_blk), lambda i: (0, i)),
        ],
        out_shape=[
            jax.ShapeDtypeStruct((_TOP_K, m_total), jnp.int32),
            jax.ShapeDtypeStruct((_TOP_K, m_total), jnp.float32),
        ],
    )(weight, flat, flat)
    return idx_t.T, val_t.T


def kernel(x, weight):
    Bx, Sx, Hx = x.shape
    flat = x.reshape(-1, Hx)
    idx, w = _router(flat, weight, 512)
    return idx.reshape(Bx, Sx, _TOP_K), w.reshape(Bx, Sx, _TOP_K)


# speedup vs baseline: 1.0021x; 1.0021x over previous
import functools
import jax
import jax.numpy as jnp
from jax.experimental import pallas as pl

_E = 64
_TOP_K = 8
_SCALE = 2.5


def _router_block(w_ref, xa_ref, xb_ref, xc_ref, xd_ref, idx_ref, val_ref):
    w = w_ref[...]
    dn = (((1,), (1,)), ((), ()))
    la = jax.lax.dot_general(w, xa_ref[...], dimension_numbers=dn,
                             preferred_element_type=jnp.float32)
    lb = jax.lax.dot_general(w, xb_ref[...], dimension_numbers=dn,
                             preferred_element_type=jnp.float32)
    lc = jax.lax.dot_general(w, xc_ref[...], dimension_numbers=dn,
                             preferred_element_type=jnp.float32)
    ld = jax.lax.dot_general(w, xd_ref[...], dimension_numbers=dn,
                             preferred_element_type=jnp.float32)
    logits = jnp.concatenate([la, lb, lc, ld], axis=1)
    iota = jax.lax.broadcasted_iota(jnp.int32, logits.shape, 0)
    work = logits
    idx_rows = []
    val_rows = []
    for k in range(_TOP_K):
        mk = jnp.max(work, axis=0, keepdims=True)
        if k == 0:
            m = mk
            denom = jnp.sum(jnp.exp(logits - m), axis=0, keepdims=True)
            inv = _SCALE / denom
        sel = jnp.min(jnp.where(work == mk, iota, _E), axis=0, keepdims=True)
        idx_rows.append(sel)
        val_rows.append(jnp.exp(mk - m) * inv)
        work = jnp.where(iota == sel, -jnp.inf, work)
    idx_ref[...] = jnp.concatenate(idx_rows, axis=0)
    val_ref[...] = jnp.concatenate(val_rows, axis=0)


@functools.partial(jax.jit, static_argnames=("m_blk",))
def _router(flat, weight, m_blk):
    m_total, h = flat.shape
    n_steps = m_total // (4 * m_blk)
    idx_t, val_t = pl.pallas_call(
        _router_block,
        grid=(n_steps,),
        in_specs=[
            pl.BlockSpec((_E, h), lambda i: (0, 0)),
            pl.BlockSpec((m_blk, h), lambda i: (4 * i, 0)),
            pl.BlockSpec((m_blk, h), lambda i: (4 * i + 1, 0)),
            pl.BlockSpec((m_blk, h), lambda i: (4 * i + 2, 0)),
            pl.BlockSpec((m_blk, h), lambda i: (4 * i + 3, 0)),
        ],
        out_specs=[
            pl.BlockSpec((_TOP_K, 4 * m_blk), lambda i: (0, i)),
            pl.BlockSpec((_TOP_K, 4 * m_blk), lambda i: (0, i)),
        ],
        out_shape=[
            jax.ShapeDtypeStruct((_TOP_K, m_total), jnp.int32),
            jax.ShapeDtypeStruct((_TOP_K, m_total), jnp.float32),
        ],
    )(weight, flat, flat, flat, flat)
    return idx_t.T, val_t.T


def kernel(x, weight):
    Bx, Sx, Hx = x.shape
    flat = x.reshape(-1, Hx)
    idx, w = _router(flat, weight, 256)
    return idx.reshape(Bx, Sx, _TOP_K), w.reshape(Bx, Sx, _TOP_K)


# speedup vs baseline: 1.0207x; 1.0185x over previous
import functools

import jax
import jax.numpy as jnp
from jax.experimental import pallas as pl
from jax.experimental.pallas import tpu as pltpu

_E = 64
_TOP_K = 8
_SCALE = 2.5


def _router_block(w_ref, x_ref, idx_ref, val_ref):
    logits = jax.lax.dot_general(
        w_ref[...], x_ref[...],
        dimension_numbers=(((1,), (1,)), ((), ())),
        preferred_element_type=jnp.float32,
    )
    iota = jax.lax.broadcasted_iota(jnp.int32, logits.shape, 0)
    work = logits
    idx_rows = []
    val_rows = []
    for k in range(_TOP_K):
        mk = jnp.max(work, axis=0, keepdims=True)
        if k == 0:
            m = mk
            denom = jnp.sum(jnp.exp(logits - m), axis=0, keepdims=True)
            inv = _SCALE / denom
        sel = jnp.min(jnp.where(work == mk, iota, _E), axis=0, keepdims=True)
        idx_rows.append(sel)
        val_rows.append(jnp.exp(mk - m) * inv)
        work = jnp.where(iota == sel, -jnp.inf, work)

    idx_ref[...] = jnp.concatenate(idx_rows, axis=0)
    val_ref[...] = jnp.concatenate(val_rows, axis=0)


@functools.partial(jax.jit, static_argnames=("m_blk",))
def _router(flat, weight, m_blk):
    m_total, h = flat.shape
    grid = (m_total // m_blk,)
    idx_t, val_t = pl.pallas_call(
        _router_block,
        grid=grid,
        in_specs=[
            pl.BlockSpec((_E, h), lambda i: (0, 0)),
            pl.BlockSpec((m_blk, h), lambda i: (i, 0)),
        ],
        out_specs=[
            pl.BlockSpec((_TOP_K, m_blk), lambda i: (0, i)),
            pl.BlockSpec((_TOP_K, m_blk), lambda i: (0, i)),
        ],
        out_shape=[
            jax.ShapeDtypeStruct((_TOP_K, m_total), jnp.int32),
            jax.ShapeDtypeStruct((_TOP_K, m_total), jnp.float32),
        ],
        compiler_params=pltpu.CompilerParams(
            dimension_semantics=(pltpu.PARALLEL,),
        ),
    )(weight, flat)
    return idx_t.T, val_t.T


def kernel(x, weight):
    Bx, Sx, Hx = x.shape
    flat = x.reshape(-1, Hx)
    idx, w = _router(flat, weight, 1024)
    return idx.reshape(Bx, Sx, _TOP_K), w.reshape(Bx, Sx, _TOP_K)


# speedup vs baseline: 1.0213x; 1.0006x over previous
import functools

import jax
import jax.numpy as jnp
from jax.experimental import pallas as pl

_E = 64
_TOP_K = 8
_SCALE = 2.5


def _router_block(w_ref, x_ref, idx_ref, val_ref):
    logits = jax.lax.dot_general(
        w_ref[...], x_ref[...],
        dimension_numbers=(((1,), (1,)), ((), ())),
        preferred_element_type=jnp.float32,
    )
    iota = jax.lax.broadcasted_iota(jnp.int32, logits.shape, 0)
    work = logits
    idx_rows = []
    val_rows = []
    for k in range(_TOP_K):
        mk = jnp.max(work, axis=0, keepdims=True)
        if k == 0:
            m = mk
            denom = jnp.sum(jnp.exp(logits - m), axis=0, keepdims=True)
            inv = _SCALE / denom
        sel = jnp.min(jnp.where(work == mk, iota, _E), axis=0, keepdims=True)
        idx_rows.append(sel)
        val_rows.append(jnp.exp(mk - m) * inv)
        work = jnp.where(iota == sel, -jnp.inf, work)

    idx_ref[...] = jnp.concatenate(idx_rows, axis=0)
    val_ref[...] = jnp.concatenate(val_rows, axis=0)


@functools.partial(jax.jit, static_argnames=("m_blk",))
def _router(flat, weight, m_blk):
    m_total, h = flat.shape
    grid = (m_total // m_blk,)
    idx_t, val_t = pl.pallas_call(
        _router_block,
        grid=grid,
        in_specs=[
            pl.BlockSpec((_E, h), lambda i: (0, 0)),
            pl.BlockSpec((m_blk, h), lambda i: (i, 0)),
        ],
        out_specs=[
            pl.BlockSpec((_TOP_K, m_blk), lambda i: (0, i)),
            pl.BlockSpec((_TOP_K, m_blk), lambda i: (0, i)),
        ],
        out_shape=[
            jax.ShapeDtypeStruct((_TOP_K, m_total), jnp.int32),
            jax.ShapeDtypeStruct((_TOP_K, m_total), jnp.float32),
        ],
    )(weight, flat)
    return idx_t.T, val_t.T


def kernel(x, weight):
    Bx, Sx, Hx = x.shape
    flat = x.reshape(-1, Hx)
    idx, w = _router(flat, weight, 1024)
    return idx.reshape(Bx, Sx, _TOP_K), w.reshape(Bx, Sx, _TOP_K)
